# baseline (speedup 1.0000x reference)
import functools

import jax
import jax.numpy as jnp
from jax import lax
from jax.experimental import pallas as pl
from jax.experimental.pallas import tpu as pltpu
from jax.experimental.pallas import tpu_sc as plsc

B = 16384
D = 64
PATHS = 100
NODES = 1000000
NC = 2
NS = 16
L = 16
NW = NC * NS
BPW = B // NW

PIECE = 1024
NP = 32
OWN = NP * PIECE
CAP = 1536
SCAP = 64
SCH = 2048
NROUND = B // SCH
DUMP = B
EXR = B + SCAP
TAIL0 = (NODES // 128) * 128
TMAXA = TAIL0 - 128

_mesh = plsc.VectorSubcoreMesh(core_axis_name="c", subcore_axis_name="s")


def _wid():
    return lax.axis_index("s") * NC + lax.axis_index("c")


def _scalar(v, h):
    lanes = lax.iota(jnp.int32, L)
    return jnp.sum(jnp.where(lanes == h, v, 0))


def _count(m):
    n16 = plsc.all_reduce_population_count(m)
    return lax.squeeze(lax.slice(n16, (0,), (1,)), (0,))


@functools.partial(
    pl.kernel,
    mesh=_mesh,
    out_type=(
        jax.ShapeDtypeStruct((EXR, 128), jnp.float32),
        jax.ShapeDtypeStruct((EXR, 128), jnp.float32),
    ),
    scratch_types=[
        pltpu.VMEM((SCH,), jnp.int32),
        pltpu.VMEM((SCH,), jnp.int32),
        pltpu.VMEM((CAP,), jnp.int32),
        pltpu.VMEM((CAP,), jnp.int32),
        pltpu.VMEM((CAP,), jnp.int32),
        pltpu.VMEM((CAP,), jnp.int32),
        pltpu.VMEM((9, D, 128), jnp.float32),
        pltpu.VMEM((SCAP, 128), jnp.float32),
        pltpu.VMEM((SCAP, 128), jnp.float32),
        pltpu.VMEM((SCAP, 128), jnp.float32),
        pltpu.VMEM((SCAP, 128), jnp.float32),
        pltpu.VMEM((SCAP,), jnp.int32),
        pltpu.VMEM((SCAP,), jnp.int32),
        pltpu.VMEM((SCAP,), jnp.int32),
        pltpu.VMEM((SCAP,), jnp.int32),
        pltpu.VMEM((L,), jnp.int32),
        pltpu.VMEM((L,), jnp.int32),
        pltpu.SemaphoreType.DMA,
        pltpu.SemaphoreType.DMA,
    ],
    compiler_params=pltpu.CompilerParams(
        needs_layout_passes=False, use_tc_tiling_on_sc=True),
)
def _extract_sc(sn_hbm, en_hbm, ntT_hbm, ntail_hbm, sex_hbm, eex_hbm,
                sbuf, ebuf, sil, sql, eil, eql, slab,
                sstA, estA, sstB, estB, sdsA, edsA, sdsB, edsB,
                tmpi, tmpq, sem, sem2):
    w = _wid()
    lo = w * OWN
    hi = lo + OWN
    lanes = lax.iota(jnp.int32, L)

    pltpu.sync_copy(ntail_hbm, slab.at[8])

    def round_body(r, tails):
        pltpu.sync_copy(sn_hbm.at[pl.ds(r * SCH, SCH)], sbuf)
        pltpu.sync_copy(en_hbm.at[pl.ds(r * SCH, SCH)], ebuf)

        def chunk_body(t, tails):
            st, et = tails
            qv = r * SCH + t * L + lanes
            sv = sbuf[pl.ds(t * L, L)]
            m = jnp.logical_and(sv >= lo, sv < hi)
            n = _count(m)
            plsc.store_compressed(sil.at[pl.ds(st, L)], sv, mask=m)
            plsc.store_compressed(sql.at[pl.ds(st, L)], qv, mask=m)
            st = jnp.minimum(st + n, CAP - L)
            ev = ebuf[pl.ds(t * L, L)]
            m = jnp.logical_and(ev >= lo, ev < hi)
            n = _count(m)
            plsc.store_compressed(eil.at[pl.ds(et, L)], ev, mask=m)
            plsc.store_compressed(eql.at[pl.ds(et, L)], qv, mask=m)
            et = jnp.minimum(et + n, CAP - L)
            return st, et

        return lax.fori_loop(0, SCH // L, chunk_body, tails)

    stail, etail = lax.fori_loop(0, NROUND, round_body, (0, 0))

    def do_piece(t, sstage, estage, sdst, edst):
        bp0 = lo + t * PIECE
        copies = []
        for k in range(8):
            bk = pl.multiple_of(jnp.minimum(bp0 + k * 128, TMAXA), 128)
            copies.append(pltpu.async_copy(
                ntT_hbm.at[:, pl.ds(bk, 128)], slab.at[k], sem))
        for c in copies:
            c.wait()

        for j in range(SCAP // L):
            dump16 = DUMP + j * L + lanes
            sdst[pl.ds(j * L, L)] = dump16
            edst[pl.ds(j * L, L)] = dump16

        def drain(ilist, qlist, tail, stage, dst):
            def chunk(ch, ptail):
                iv = ilist[pl.ds(ch * L, L)]
                qv = qlist[pl.ds(ch * L, L)]
                valid = (ch * L + lanes) < tail
                m = jnp.logical_and(valid, jnp.logical_and(
                    iv >= bp0, iv < bp0 + PIECE))
                n = _count(m)
                plsc.store_compressed(tmpi.at[pl.ds(0, L)], iv, mask=m)
                plsc.store_compressed(tmpq.at[pl.ds(0, L)], qv, mask=m)
                plsc.store_compressed(dst.at[pl.ds(ptail, L)], qv, mask=m)
                tiv = tmpi[...]

                def hit(h, slot):
                    i_s = _scalar(tiv, h)
                    off = i_s - bp0
                    k = lax.shift_right_logical(off, 7)
                    bkc = jnp.minimum(bp0 + k * 128, TMAXA)
                    is_tail = i_s >= TAIL0
                    k = jnp.where(is_tail, 8, k)
                    lane = jnp.where(is_tail, i_s - TAIL0, i_s - bkc)
                    kf = jnp.full((L,), k, jnp.int32)
                    lf = jnp.full((L,), lane, jnp.int32)
                    for c in range(D // L):
                        g = plsc.load_gather(
                            slab, [kf, lanes + c * L, lf])
                        stage[slot, pl.ds(c * L, L)] = g
                    return slot + 1

                ptail = lax.fori_loop(0, n, hit, ptail)
                return jnp.minimum(ptail, SCAP - L)

            nch = lax.div(tail + (L - 1), L)
            return lax.fori_loop(0, nch, chunk, 0)

        drain(sil, sql, stail, sstage, sdst)
        drain(eil, eql, etail, estage, edst)
        cs = pltpu.async_copy(sstage, sex_hbm.at[sdst], sem2)
        ce = pltpu.async_copy(estage, eex_hbm.at[edst], sem2)
        return cs, ce

    def pair_body(j, carry):
        ca = do_piece(2 * j, sstA, estA, sdsA, edsA)
        cb = do_piece(2 * j + 1, sstB, estB, sdsB, edsB)
        for c in ca + cb:
            c.wait()
        return carry

    lax.fori_loop(0, NP // 2, pair_body, 0)


@functools.partial(
    pl.kernel,
    mesh=_mesh,
    out_type=jax.ShapeDtypeStruct((B,), jnp.float32),
    scratch_types=[
        pltpu.VMEM((BPW,), jnp.int32),
        pltpu.VMEM((PATHS, 128), jnp.float32),
        pltpu.VMEM((128, 128), jnp.float32),
        pltpu.VMEM((128, 128), jnp.float32),
        pltpu.VMEM((BPW,), jnp.float32),
        pltpu.SemaphoreType.DMA,
    ],
    compiler_params=pltpu.CompilerParams(
        needs_layout_passes=False, use_tc_tiling_on_sc=True),
)
def _pair_sc(sex_hbm, eex_hbm, pt_hbm, ptab_hbm, out_hbm,
             pidx, ptab, srows, erows, outv, sem):
    w = _wid()
    base = w * BPW
    lanes = lax.iota(jnp.int32, L)

    pltpu.sync_copy(pt_hbm.at[pl.ds(base, BPW)], pidx)
    pltpu.sync_copy(ptab_hbm, ptab)

    def sub_body(sb, carry):
        qb = base + sb * 128
        ca = pltpu.async_copy(sex_hbm.at[pl.ds(qb, 128)], srows, sem)
        cb = pltpu.async_copy(eex_hbm.at[pl.ds(qb, 128)], erows, sem)
        ca.wait()
        cb.wait()

        def group_body(g, carry):
            pvec = pidx[pl.ds(sb * 128 + g * L, L)]
            out16 = jnp.zeros((L,), jnp.float32)
            for k in range(L):
                pid = lax.squeeze(lax.slice(pvec, (k,), (k + 1,)), (0,))
                row = g * L + k
                acc = jnp.zeros((L,), jnp.float32)
                for c in range(D // L):
                    sl = pl.ds(c * L, L)
                    pvv = ptab[pid, sl]
                    svv = srows[row, sl]
                    evv = erows[row, sl]
                    acc = acc + jnp.where(pvv > 0.0, svv * evv, 0.0)
                tot = jnp.sum(acc)
                out16 = jnp.where(lanes == k, tot, out16)
            z = jnp.exp(-jnp.abs(out16))
            sig = jnp.where(out16 >= 0.0, 1.0 / (1.0 + z), z / (1.0 + z))
            outv[pl.ds(sb * 128 + g * L, L)] = sig
            return carry

        return lax.fori_loop(0, 128 // L, group_body, carry)

    lax.fori_loop(0, BPW // 128, sub_body, 0)

    pltpu.sync_copy(outv, out_hbm.at[pl.ds(base, BPW)])


def kernel(start_node, end_node, path, node_table, path_table):
    sn = start_node.astype(jnp.int32)
    en = end_node.astype(jnp.int32)
    pt = path.astype(jnp.int32)
    nt_tail = jnp.pad(node_table[TAIL0:].T, ((0, 0), (0, 128 - (NODES - TAIL0))))
    sex, eex = _extract_sc(sn, en, node_table.T, nt_tail)
    ptab_pad = jnp.pad(path_table, ((0, 0), (0, 128 - D)))
    return _pair_sc(sex, eex, pt, ptab_pad)


# speedup vs baseline: 1.1026x; 1.1026x over previous
import functools

import jax
import jax.numpy as jnp
from jax import lax
from jax.experimental import pallas as pl
from jax.experimental.pallas import tpu as pltpu
from jax.experimental.pallas import tpu_sc as plsc

B = 16384
D = 64
PATHS = 100
NODES = 1000000
NC = 2
NS = 16
L = 16
NW = NC * NS
BPW = B // NW

PIECE = 1024
NP = 32
OWN = NP * PIECE
CAP = 1536
SCAP = 64
SCH = 2048
NROUND = B // SCH
DUMP = B
EXR = B + SCAP
TAIL0 = (NODES // 128) * 128
TMAXA = TAIL0 - 128

_mesh = plsc.VectorSubcoreMesh(core_axis_name="c", subcore_axis_name="s")


def _wid():
    return lax.axis_index("s") * NC + lax.axis_index("c")


def _scalar(v, h):
    lanes = lax.iota(jnp.int32, L)
    return jnp.sum(jnp.where(lanes == h, v, 0))


def _count(m):
    n16 = plsc.all_reduce_population_count(m)
    return lax.squeeze(lax.slice(n16, (0,), (1,)), (0,))


@functools.partial(
    pl.kernel,
    mesh=_mesh,
    out_type=(
        jax.ShapeDtypeStruct((EXR, 128), jnp.float32),
        jax.ShapeDtypeStruct((EXR, 128), jnp.float32),
    ),
    scratch_types=[
        pltpu.VMEM((SCH,), jnp.int32),
        pltpu.VMEM((SCH,), jnp.int32),
        pltpu.VMEM((CAP,), jnp.int32),
        pltpu.VMEM((CAP,), jnp.int32),
        pltpu.VMEM((CAP,), jnp.int32),
        pltpu.VMEM((CAP,), jnp.int32),
        pltpu.VMEM((9, D, 128), jnp.float32),
        pltpu.VMEM((SCAP, 128), jnp.float32),
        pltpu.VMEM((SCAP, 128), jnp.float32),
        pltpu.VMEM((SCAP, 128), jnp.float32),
        pltpu.VMEM((SCAP, 128), jnp.float32),
        pltpu.VMEM((SCAP,), jnp.int32),
        pltpu.VMEM((SCAP,), jnp.int32),
        pltpu.VMEM((SCAP,), jnp.int32),
        pltpu.VMEM((SCAP,), jnp.int32),
        pltpu.VMEM((L,), jnp.int32),
        pltpu.VMEM((L,), jnp.int32),
        pltpu.SemaphoreType.DMA,
        pltpu.SemaphoreType.DMA,
    ],
    compiler_params=pltpu.CompilerParams(
        needs_layout_passes=False, use_tc_tiling_on_sc=True),
)
def _extract_sc(sn_hbm, en_hbm, ntT_hbm, ntail_hbm, sex_hbm, eex_hbm,
                sbuf, ebuf, sil, sql, eil, eql, slab,
                sstA, estA, sstB, estB, sdsA, edsA, sdsB, edsB,
                tmpi, tmpq, sem, sem2):
    w = _wid()
    lo = w * OWN
    hi = lo + OWN
    lanes = lax.iota(jnp.int32, L)

    pltpu.sync_copy(ntail_hbm, slab.at[8])

    def round_body(r, tails):
        pltpu.sync_copy(sn_hbm.at[pl.ds(r * SCH, SCH)], sbuf)
        pltpu.sync_copy(en_hbm.at[pl.ds(r * SCH, SCH)], ebuf)

        def chunk_body(t, tails):
            st, et = tails
            qv = r * SCH + t * L + lanes
            sv = sbuf[pl.ds(t * L, L)]
            m = jnp.logical_and(sv >= lo, sv < hi)
            n = _count(m)
            plsc.store_compressed(sil.at[pl.ds(st, L)], sv, mask=m)
            plsc.store_compressed(sql.at[pl.ds(st, L)], qv, mask=m)
            st = jnp.minimum(st + n, CAP - L)
            ev = ebuf[pl.ds(t * L, L)]
            m = jnp.logical_and(ev >= lo, ev < hi)
            n = _count(m)
            plsc.store_compressed(eil.at[pl.ds(et, L)], ev, mask=m)
            plsc.store_compressed(eql.at[pl.ds(et, L)], qv, mask=m)
            et = jnp.minimum(et + n, CAP - L)
            return st, et

        return lax.fori_loop(0, SCH // L, chunk_body, tails)

    stail, etail = lax.fori_loop(0, NROUND, round_body, (0, 0))

    def do_piece(t, sstage, estage, sdst, edst):
        bp0 = lo + t * PIECE
        copies = []
        for k in range(8):
            bk = pl.multiple_of(jnp.minimum(bp0 + k * 128, TMAXA), 128)
            copies.append(pltpu.async_copy(
                ntT_hbm.at[:, pl.ds(bk, 128)], slab.at[k], sem))
        for c in copies:
            c.wait()

        for j in range(SCAP // L):
            dump16 = DUMP + j * L + lanes
            sdst[pl.ds(j * L, L)] = dump16
            edst[pl.ds(j * L, L)] = dump16

        def drain(ilist, qlist, tail, stage, dst):
            def chunk(ch, ptail):
                iv = ilist[pl.ds(ch * L, L)]
                qv = qlist[pl.ds(ch * L, L)]
                valid = (ch * L + lanes) < tail
                m = jnp.logical_and(valid, jnp.logical_and(
                    iv >= bp0, iv < bp0 + PIECE))
                n = _count(m)
                plsc.store_compressed(tmpi.at[pl.ds(0, L)], iv, mask=m)
                plsc.store_compressed(tmpq.at[pl.ds(0, L)], qv, mask=m)
                plsc.store_compressed(dst.at[pl.ds(ptail, L)], qv, mask=m)
                tiv = tmpi[...]

                def hit(h, slot):
                    i_s = _scalar(tiv, h)
                    off = i_s - bp0
                    k = lax.shift_right_logical(off, 7)
                    bkc = jnp.minimum(bp0 + k * 128, TMAXA)
                    is_tail = i_s >= TAIL0
                    k = jnp.where(is_tail, 8, k)
                    lane = jnp.where(is_tail, i_s - TAIL0, i_s - bkc)
                    kf = jnp.full((L,), k, jnp.int32)
                    lf = jnp.full((L,), lane, jnp.int32)
                    for c in range(D // L):
                        g = plsc.load_gather(
                            slab, [kf, lanes + c * L, lf])
                        stage[slot, pl.ds(c * L, L)] = g
                    return slot + 1

                ptail = lax.fori_loop(0, n, hit, ptail)
                return jnp.minimum(ptail, SCAP - L)

            nch = lax.div(tail + (L - 1), L)
            return lax.fori_loop(0, nch, chunk, 0)

        ps = drain(sil, sql, stail, sstage, sdst)
        pe = drain(eil, eql, etail, estage, edst)
        for c in range(SCAP // L):
            csl = pl.ds(c * L, L)

            @pl.when(ps > c * L)
            def _():
                pltpu.async_copy(
                    sstage.at[csl], sex_hbm.at[sdst.at[csl]], sem2).wait()

            @pl.when(pe > c * L)
            def _():
                pltpu.async_copy(
                    estage.at[csl], eex_hbm.at[edst.at[csl]], sem2).wait()

    def piece_body(t, carry):
        do_piece(t, sstA, estA, sdsA, edsA)
        return carry

    lax.fori_loop(0, NP, piece_body, 0)


@functools.partial(
    pl.kernel,
    mesh=_mesh,
    out_type=jax.ShapeDtypeStruct((B,), jnp.float32),
    scratch_types=[
        pltpu.VMEM((BPW,), jnp.int32),
        pltpu.VMEM((PATHS, 128), jnp.float32),
        pltpu.VMEM((128, 128), jnp.float32),
        pltpu.VMEM((128, 128), jnp.float32),
        pltpu.VMEM((BPW,), jnp.float32),
        pltpu.SemaphoreType.DMA,
    ],
    compiler_params=pltpu.CompilerParams(
        needs_layout_passes=False, use_tc_tiling_on_sc=True),
)
def _pair_sc(sex_hbm, eex_hbm, pt_hbm, ptab_hbm, out_hbm,
             pidx, ptab, srows, erows, outv, sem):
    w = _wid()
    base = w * BPW
    lanes = lax.iota(jnp.int32, L)

    pltpu.sync_copy(pt_hbm.at[pl.ds(base, BPW)], pidx)
    pltpu.sync_copy(ptab_hbm, ptab)

    def sub_body(sb, carry):
        qb = base + sb * 128
        ca = pltpu.async_copy(sex_hbm.at[pl.ds(qb, 128)], srows, sem)
        cb = pltpu.async_copy(eex_hbm.at[pl.ds(qb, 128)], erows, sem)
        ca.wait()
        cb.wait()

        def group_body(g, carry):
            pvec = pidx[pl.ds(sb * 128 + g * L, L)]
            out16 = jnp.zeros((L,), jnp.float32)
            for k in range(L):
                pid = lax.squeeze(lax.slice(pvec, (k,), (k + 1,)), (0,))
                row = g * L + k
                acc = jnp.zeros((L,), jnp.float32)
                for c in range(D // L):
                    sl = pl.ds(c * L, L)
                    pvv = ptab[pid, sl]
                    svv = srows[row, sl]
                    evv = erows[row, sl]
                    acc = acc + jnp.where(pvv > 0.0, svv * evv, 0.0)
                tot = jnp.sum(acc)
                out16 = jnp.where(lanes == k, tot, out16)
            z = jnp.exp(-jnp.abs(out16))
            sig = jnp.where(out16 >= 0.0, 1.0 / (1.0 + z), z / (1.0 + z))
            outv[pl.ds(sb * 128 + g * L, L)] = sig
            return carry

        return lax.fori_loop(0, 128 // L, group_body, carry)

    lax.fori_loop(0, BPW // 128, sub_body, 0)

    pltpu.sync_copy(outv, out_hbm.at[pl.ds(base, BPW)])


def kernel(start_node, end_node, path, node_table, path_table):
    sn = start_node.astype(jnp.int32)
    en = end_node.astype(jnp.int32)
    pt = path.astype(jnp.int32)
    nt_tail = jnp.pad(node_table[TAIL0:].T, ((0, 0), (0, 128 - (NODES - TAIL0))))
    sex, eex = _extract_sc(sn, en, node_table.T, nt_tail)
    ptab_pad = jnp.pad(path_table, ((0, 0), (0, 128 - D)))
    return _pair_sc(sex, eex, pt, ptab_pad)


# speedup vs baseline: 1.1752x; 1.0659x over previous
import functools

import jax
import jax.numpy as jnp
from jax import lax
from jax.experimental import pallas as pl
from jax.experimental.pallas import tpu as pltpu
from jax.experimental.pallas import tpu_sc as plsc

B = 16384
D = 64
PATHS = 100
NODES = 1000000
NC = 2
NS = 16
L = 16
NW = NC * NS
BPW = B // NW

PIECE = 1024
NP = 32
OWN = NP * PIECE
CAP = 1536
SCAP = 64
SCH = 4096
NROUND = B // SCH
DUMP = B
EXR = B + SCAP
TAIL0 = (NODES // 128) * 128
TMAXA = TAIL0 - 128

_mesh = plsc.VectorSubcoreMesh(core_axis_name="c", subcore_axis_name="s")


def _wid():
    return lax.axis_index("s") * NC + lax.axis_index("c")


def _scalar(v, h):
    lanes = lax.iota(jnp.int32, L)
    return jnp.sum(jnp.where(lanes == h, v, 0))


def _count(m):
    n16 = plsc.all_reduce_population_count(m)
    return lax.squeeze(lax.slice(n16, (0,), (1,)), (0,))


@functools.partial(
    pl.kernel,
    mesh=_mesh,
    out_type=(
        jax.ShapeDtypeStruct((EXR, 128), jnp.float32),
        jax.ShapeDtypeStruct((EXR, 128), jnp.float32),
    ),
    scratch_types=[
        pltpu.VMEM((SCH,), jnp.int32),
        pltpu.VMEM((SCH,), jnp.int32),
        pltpu.VMEM((CAP,), jnp.int32),
        pltpu.VMEM((CAP,), jnp.int32),
        pltpu.VMEM((CAP,), jnp.int32),
        pltpu.VMEM((CAP,), jnp.int32),
        pltpu.VMEM((9, D, 128), jnp.float32),
        pltpu.VMEM((SCAP, 128), jnp.float32),
        pltpu.VMEM((SCAP, 128), jnp.float32),
        pltpu.VMEM((SCAP, 128), jnp.float32),
        pltpu.VMEM((SCAP, 128), jnp.float32),
        pltpu.VMEM((SCAP,), jnp.int32),
        pltpu.VMEM((SCAP,), jnp.int32),
        pltpu.VMEM((SCAP,), jnp.int32),
        pltpu.VMEM((SCAP,), jnp.int32),
        pltpu.VMEM((L,), jnp.int32),
        pltpu.VMEM((L,), jnp.int32),
        pltpu.SemaphoreType.DMA,
        pltpu.SemaphoreType.DMA,
    ],
    compiler_params=pltpu.CompilerParams(
        needs_layout_passes=False, use_tc_tiling_on_sc=True),
)
def _extract_sc(sn_hbm, en_hbm, ntT_hbm, ntail_hbm, sex_hbm, eex_hbm,
                sbuf, ebuf, sil, sql, eil, eql, slab,
                sstA, estA, sstB, estB, sdsA, edsA, sdsB, edsB,
                tmpi, tmpq, sem, sem2):
    w = _wid()
    lo = w * OWN
    hi = lo + OWN
    lanes = lax.iota(jnp.int32, L)

    pltpu.sync_copy(ntail_hbm, slab.at[8])

    def round_body(r, tails):
        ca = pltpu.async_copy(sn_hbm.at[pl.ds(r * SCH, SCH)], sbuf, sem)
        cb = pltpu.async_copy(en_hbm.at[pl.ds(r * SCH, SCH)], ebuf, sem)
        ca.wait()
        cb.wait()

        def chunk_body(t, tails):
            st, et = tails
            qv = r * SCH + t * L + lanes
            sv = sbuf[pl.ds(t * L, L)]
            m = jnp.logical_and(sv >= lo, sv < hi)
            n = _count(m)
            plsc.store_compressed(sil.at[pl.ds(st, L)], sv, mask=m)
            plsc.store_compressed(sql.at[pl.ds(st, L)], qv, mask=m)
            st = jnp.minimum(st + n, CAP - L)
            ev = ebuf[pl.ds(t * L, L)]
            m = jnp.logical_and(ev >= lo, ev < hi)
            n = _count(m)
            plsc.store_compressed(eil.at[pl.ds(et, L)], ev, mask=m)
            plsc.store_compressed(eql.at[pl.ds(et, L)], qv, mask=m)
            et = jnp.minimum(et + n, CAP - L)
            return st, et

        return lax.fori_loop(0, SCH // L, chunk_body, tails)

    stail, etail = lax.fori_loop(0, NROUND, round_body, (0, 0))

    def do_piece(t, sstage, estage, sdst, edst):
        bp0 = lo + t * PIECE
        copies = []
        for k in range(8):
            bk = pl.multiple_of(jnp.minimum(bp0 + k * 128, TMAXA), 128)
            copies.append(pltpu.async_copy(
                ntT_hbm.at[:, pl.ds(bk, 128)], slab.at[k], sem))
        for c in copies:
            c.wait()

        for j in range(SCAP // L):
            dump16 = DUMP + j * L + lanes
            sdst[pl.ds(j * L, L)] = dump16
            edst[pl.ds(j * L, L)] = dump16

        def drain(ilist, qlist, tail, stage, dst):
            def chunk(ch, ptail):
                iv = ilist[pl.ds(ch * L, L)]
                qv = qlist[pl.ds(ch * L, L)]
                valid = (ch * L + lanes) < tail
                m = jnp.logical_and(valid, jnp.logical_and(
                    iv >= bp0, iv < bp0 + PIECE))
                n = _count(m)
                plsc.store_compressed(tmpi.at[pl.ds(0, L)], iv, mask=m)
                plsc.store_compressed(tmpq.at[pl.ds(0, L)], qv, mask=m)
                plsc.store_compressed(dst.at[pl.ds(ptail, L)], qv, mask=m)
                tiv = tmpi[...]

                def hit(h, slot):
                    i_s = _scalar(tiv, h)
                    off = i_s - bp0
                    k = lax.shift_right_logical(off, 7)
                    bkc = jnp.minimum(bp0 + k * 128, TMAXA)
                    is_tail = i_s >= TAIL0
                    k = jnp.where(is_tail, 8, k)
                    lane = jnp.where(is_tail, i_s - TAIL0, i_s - bkc)
                    kf = jnp.full((L,), k, jnp.int32)
                    lf = jnp.full((L,), lane, jnp.int32)
                    for c in range(D // L):
                        g = plsc.load_gather(
                            slab, [kf, lanes + c * L, lf])
                        stage[slot, pl.ds(c * L, L)] = g
                    return slot + 1

                ptail = lax.fori_loop(0, n, hit, ptail)
                return jnp.minimum(ptail, SCAP - L)

            nch = lax.div(tail + (L - 1), L)
            return lax.fori_loop(0, nch, chunk, 0)

        ps = drain(sil, sql, stail, sstage, sdst)
        pe = drain(eil, eql, etail, estage, edst)
        for c in range(SCAP // L):
            csl = pl.ds(c * L, L)

            @pl.when(ps > c * L)
            def _():
                pltpu.async_copy(
                    sstage.at[csl], sex_hbm.at[sdst.at[csl]], sem2)

            @pl.when(pe > c * L)
            def _():
                pltpu.async_copy(
                    estage.at[csl], eex_hbm.at[edst.at[csl]], sem2)

        return lax.div(ps + (L - 1), L) + lax.div(pe + (L - 1), L)

    def drain_pending(n):
        def one(i, c):
            pltpu.make_async_copy(
                sex_hbm.at[pl.ds(0, L)], sstA.at[pl.ds(0, L)], sem2).wait()
            return c

        lax.fori_loop(0, n, one, 0)

    def pair_body(j, pends):
        pA, pB = pends
        drain_pending(pA)
        pA = do_piece(2 * j, sstA, estA, sdsA, edsA)
        drain_pending(pB)
        pB = do_piece(2 * j + 1, sstB, estB, sdsB, edsB)
        return pA, pB

    pA, pB = lax.fori_loop(0, NP // 2, pair_body, (0, 0))
    drain_pending(pA + pB)


@functools.partial(
    pl.kernel,
    mesh=_mesh,
    out_type=jax.ShapeDtypeStruct((B,), jnp.float32),
    scratch_types=[
        pltpu.VMEM((BPW,), jnp.int32),
        pltpu.VMEM((PATHS, 128), jnp.float32),
        pltpu.VMEM((128, 128), jnp.float32),
        pltpu.VMEM((128, 128), jnp.float32),
        pltpu.VMEM((BPW,), jnp.float32),
        pltpu.SemaphoreType.DMA,
    ],
    compiler_params=pltpu.CompilerParams(
        needs_layout_passes=False, use_tc_tiling_on_sc=True),
)
def _pair_sc(sex_hbm, eex_hbm, pt_hbm, ptab_hbm, out_hbm,
             pidx, ptab, srows, erows, outv, sem):
    w = _wid()
    base = w * BPW
    lanes = lax.iota(jnp.int32, L)

    pltpu.sync_copy(pt_hbm.at[pl.ds(base, BPW)], pidx)
    pltpu.sync_copy(ptab_hbm, ptab)

    def sub_body(sb, carry):
        qb = base + sb * 128
        ca = pltpu.async_copy(sex_hbm.at[pl.ds(qb, 128)], srows, sem)
        cb = pltpu.async_copy(eex_hbm.at[pl.ds(qb, 128)], erows, sem)
        ca.wait()
        cb.wait()

        def group_body(g, carry):
            pvec = pidx[pl.ds(sb * 128 + g * L, L)]
            out16 = jnp.zeros((L,), jnp.float32)
            for k in range(L):
                pid = lax.squeeze(lax.slice(pvec, (k,), (k + 1,)), (0,))
                row = g * L + k
                acc = jnp.zeros((L,), jnp.float32)
                for c in range(D // L):
                    sl = pl.ds(c * L, L)
                    pvv = ptab[pid, sl]
                    svv = srows[row, sl]
                    evv = erows[row, sl]
                    acc = acc + jnp.where(pvv > 0.0, svv * evv, 0.0)
                tot = jnp.sum(acc)
                out16 = jnp.where(lanes == k, tot, out16)
            z = jnp.exp(-jnp.abs(out16))
            sig = jnp.where(out16 >= 0.0, 1.0 / (1.0 + z), z / (1.0 + z))
            outv[pl.ds(sb * 128 + g * L, L)] = sig
            return carry

        return lax.fori_loop(0, 128 // L, group_body, carry)

    lax.fori_loop(0, BPW // 128, sub_body, 0)

    pltpu.sync_copy(outv, out_hbm.at[pl.ds(base, BPW)])


def kernel(start_node, end_node, path, node_table, path_table):
    sn = start_node.astype(jnp.int32)
    en = end_node.astype(jnp.int32)
    pt = path.astype(jnp.int32)
    nt_tail = jnp.pad(node_table[TAIL0:].T, ((0, 0), (0, 128 - (NODES - TAIL0))))
    sex, eex = _extract_sc(sn, en, node_table.T, nt_tail)
    ptab_pad = jnp.pad(path_table, ((0, 0), (0, 128 - D)))
    return _pair_sc(sex, eex, pt, ptab_pad)


# speedup vs baseline: 1.1760x; 1.0007x over previous
import functools

import jax
import jax.numpy as jnp
from jax import lax
from jax.experimental import pallas as pl
from jax.experimental.pallas import tpu as pltpu
from jax.experimental.pallas import tpu_sc as plsc

B = 16384
D = 64
PATHS = 100
NODES = 1000000
NC = 2
NS = 16
L = 16
NW = NC * NS
BPW = B // NW

PIECE = 1024
NP = 32
OWN = NP * PIECE
CAP = 1536
SCAP = 64
SCH = 4096
NROUND = B // SCH
DUMP = B
EXR = B + SCAP
TAIL0 = (NODES // 128) * 128
TMAXA = TAIL0 - 128

_mesh = plsc.VectorSubcoreMesh(core_axis_name="c", subcore_axis_name="s")


def _wid():
    return lax.axis_index("s") * NC + lax.axis_index("c")


def _scalar(v, h):
    lanes = lax.iota(jnp.int32, L)
    return jnp.sum(jnp.where(lanes == h, v, 0))


def _count(m):
    n16 = plsc.all_reduce_population_count(m)
    return lax.squeeze(lax.slice(n16, (0,), (1,)), (0,))


@functools.partial(
    pl.kernel,
    mesh=_mesh,
    out_type=(
        jax.ShapeDtypeStruct((EXR, 128), jnp.float32),
        jax.ShapeDtypeStruct((EXR, 128), jnp.float32),
    ),
    scratch_types=[
        pltpu.VMEM((SCH,), jnp.int32),
        pltpu.VMEM((SCH,), jnp.int32),
        pltpu.VMEM((CAP,), jnp.int32),
        pltpu.VMEM((CAP,), jnp.int32),
        pltpu.VMEM((CAP,), jnp.int32),
        pltpu.VMEM((CAP,), jnp.int32),
        pltpu.VMEM((9, D, 128), jnp.float32),
        pltpu.VMEM((SCAP, 128), jnp.float32),
        pltpu.VMEM((SCAP, 128), jnp.float32),
        pltpu.VMEM((SCAP, 128), jnp.float32),
        pltpu.VMEM((SCAP, 128), jnp.float32),
        pltpu.VMEM((SCAP,), jnp.int32),
        pltpu.VMEM((SCAP,), jnp.int32),
        pltpu.VMEM((SCAP,), jnp.int32),
        pltpu.VMEM((SCAP,), jnp.int32),
        pltpu.VMEM((L,), jnp.int32),
        pltpu.VMEM((L,), jnp.int32),
        pltpu.SemaphoreType.DMA,
        pltpu.SemaphoreType.DMA,
    ],
    compiler_params=pltpu.CompilerParams(
        needs_layout_passes=False, use_tc_tiling_on_sc=True),
)
def _extract_sc(sn_hbm, en_hbm, ntT_hbm, ntail_hbm, sex_hbm, eex_hbm,
                sbuf, ebuf, sil, sql, eil, eql, slab,
                sstA, estA, sstB, estB, sdsA, edsA, sdsB, edsB,
                tmpi, tmpq, sem, sem2):
    w = _wid()
    lo = w * OWN
    hi = lo + OWN
    lanes = lax.iota(jnp.int32, L)

    pltpu.sync_copy(ntail_hbm, slab.at[8])

    def round_body(r, tails):
        ca = pltpu.async_copy(sn_hbm.at[pl.ds(r * SCH, SCH)], sbuf, sem)
        cb = pltpu.async_copy(en_hbm.at[pl.ds(r * SCH, SCH)], ebuf, sem)
        ca.wait()
        cb.wait()

        def chunk_body(t, tails):
            st, et = tails
            qv = r * SCH + t * L + lanes
            sv = sbuf[pl.ds(t * L, L)]
            m = jnp.logical_and(sv >= lo, sv < hi)
            n = _count(m)
            plsc.store_compressed(sil.at[pl.ds(st, L)], sv, mask=m)
            plsc.store_compressed(sql.at[pl.ds(st, L)], qv, mask=m)
            st = jnp.minimum(st + n, CAP - L)
            ev = ebuf[pl.ds(t * L, L)]
            m = jnp.logical_and(ev >= lo, ev < hi)
            n = _count(m)
            plsc.store_compressed(eil.at[pl.ds(et, L)], ev, mask=m)
            plsc.store_compressed(eql.at[pl.ds(et, L)], qv, mask=m)
            et = jnp.minimum(et + n, CAP - L)
            return st, et

        return lax.fori_loop(0, SCH // L, chunk_body, tails)

    stail, etail = lax.fori_loop(0, NROUND, round_body, (0, 0))

    def do_piece(t, sstage, estage, sdst, edst):
        bp0 = lo + t * PIECE
        copies = []
        for k in range(8):
            bk = pl.multiple_of(jnp.minimum(bp0 + k * 128, TMAXA), 128)
            copies.append(pltpu.async_copy(
                ntT_hbm.at[:, pl.ds(bk, 128)], slab.at[k], sem))
        for c in copies:
            c.wait()

        for j in range(SCAP // L):
            dump16 = DUMP + j * L + lanes
            sdst[pl.ds(j * L, L)] = dump16
            edst[pl.ds(j * L, L)] = dump16

        def drain(ilist, qlist, tail, stage, dst):
            def chunk(ch, ptail):
                iv = ilist[pl.ds(ch * L, L)]
                qv = qlist[pl.ds(ch * L, L)]
                valid = (ch * L + lanes) < tail
                m = jnp.logical_and(valid, jnp.logical_and(
                    iv >= bp0, iv < bp0 + PIECE))
                n = _count(m)
                packv = (iv - bp0) * 16384 + qv
                plsc.store_compressed(tmpi.at[pl.ds(0, L)], packv, mask=m)
                plsc.store_compressed(dst.at[pl.ds(ptail, L)], qv, mask=m)
                tiv = tmpi[...]

                def hit(h, slot):
                    p_s = _scalar(tiv, h)
                    off = lax.shift_right_logical(p_s, 14)
                    i_s = bp0 + off
                    k = lax.shift_right_logical(off, 7)
                    bkc = jnp.minimum(bp0 + k * 128, TMAXA)
                    is_tail = i_s >= TAIL0
                    k = jnp.where(is_tail, 8, k)
                    lane = jnp.where(is_tail, i_s - TAIL0, i_s - bkc)
                    kf = jnp.full((L,), k, jnp.int32)
                    lf = jnp.full((L,), lane, jnp.int32)
                    for c in range(D // L):
                        g = plsc.load_gather(
                            slab, [kf, lanes + c * L, lf])
                        stage[slot, pl.ds(c * L, L)] = g
                    return slot + 1

                ptail = lax.fori_loop(0, n, hit, ptail)
                return jnp.minimum(ptail, SCAP - L)

            nch = lax.div(tail + (L - 1), L)
            return lax.fori_loop(0, nch, chunk, 0)

        ps = drain(sil, sql, stail, sstage, sdst)
        pe = drain(eil, eql, etail, estage, edst)
        for c in range(SCAP // L):
            csl = pl.ds(c * L, L)

            @pl.when(ps > c * L)
            def _():
                pltpu.async_copy(
                    sstage.at[csl], sex_hbm.at[sdst.at[csl]], sem2)

            @pl.when(pe > c * L)
            def _():
                pltpu.async_copy(
                    estage.at[csl], eex_hbm.at[edst.at[csl]], sem2)

        return lax.div(ps + (L - 1), L) + lax.div(pe + (L - 1), L)

    def drain_pending(n):
        def one(i, c):
            pltpu.make_async_copy(
                sex_hbm.at[pl.ds(0, L)], sstA.at[pl.ds(0, L)], sem2).wait()
            return c

        lax.fori_loop(0, n, one, 0)

    def pair_body(j, pends):
        pA, pB = pends
        drain_pending(pA)
        pA = do_piece(2 * j, sstA, estA, sdsA, edsA)
        drain_pending(pB)
        pB = do_piece(2 * j + 1, sstB, estB, sdsB, edsB)
        return pA, pB

    pA, pB = lax.fori_loop(0, NP // 2, pair_body, (0, 0))
    drain_pending(pA + pB)


@functools.partial(
    pl.kernel,
    mesh=_mesh,
    out_type=jax.ShapeDtypeStruct((B,), jnp.float32),
    scratch_types=[
        pltpu.VMEM((BPW,), jnp.int32),
        pltpu.VMEM((PATHS, 128), jnp.float32),
        pltpu.VMEM((128, 128), jnp.float32),
        pltpu.VMEM((128, 128), jnp.float32),
        pltpu.VMEM((BPW,), jnp.float32),
        pltpu.SemaphoreType.DMA,
    ],
    compiler_params=pltpu.CompilerParams(
        needs_layout_passes=False, use_tc_tiling_on_sc=True),
)
def _pair_sc(sex_hbm, eex_hbm, pt_hbm, ptab_hbm, out_hbm,
             pidx, ptab, srows, erows, outv, sem):
    w = _wid()
    base = w * BPW
    lanes = lax.iota(jnp.int32, L)

    pltpu.sync_copy(pt_hbm.at[pl.ds(base, BPW)], pidx)
    pltpu.sync_copy(ptab_hbm, ptab)

    def sub_body(sb, carry):
        qb = base + sb * 128
        ca = pltpu.async_copy(sex_hbm.at[pl.ds(qb, 128)], srows, sem)
        cb = pltpu.async_copy(eex_hbm.at[pl.ds(qb, 128)], erows, sem)
        ca.wait()
        cb.wait()

        def group_body(g, carry):
            pvec = pidx[pl.ds(sb * 128 + g * L, L)]
            out16 = jnp.zeros((L,), jnp.float32)
            for k in range(L):
                pid = lax.squeeze(lax.slice(pvec, (k,), (k + 1,)), (0,))
                row = g * L + k
                acc = jnp.zeros((L,), jnp.float32)
                for c in range(D // L):
                    sl = pl.ds(c * L, L)
                    pvv = ptab[pid, sl]
                    svv = srows[row, sl]
                    evv = erows[row, sl]
                    acc = acc + jnp.where(pvv > 0.0, svv * evv, 0.0)
                tot = jnp.sum(acc)
                out16 = jnp.where(lanes == k, tot, out16)
            z = jnp.exp(-jnp.abs(out16))
            sig = jnp.where(out16 >= 0.0, 1.0 / (1.0 + z), z / (1.0 + z))
            outv[pl.ds(sb * 128 + g * L, L)] = sig
            return carry

        return lax.fori_loop(0, 128 // L, group_body, carry)

    lax.fori_loop(0, BPW // 128, sub_body, 0)

    pltpu.sync_copy(outv, out_hbm.at[pl.ds(base, BPW)])


def kernel(start_node, end_node, path, node_table, path_table):
    sn = start_node.astype(jnp.int32)
    en = end_node.astype(jnp.int32)
    pt = path.astype(jnp.int32)
    nt_tail = jnp.pad(node_table[TAIL0:].T, ((0, 0), (0, 128 - (NODES - TAIL0))))
    sex, eex = _extract_sc(sn, en, node_table.T, nt_tail)
    ptab_pad = jnp.pad(path_table, ((0, 0), (0, 128 - D)))
    return _pair_sc(sex, eex, pt, ptab_pad)
